# Initial kernel scaffold; baseline (speedup 1.0000x reference)
import jax
import jax.numpy as jnp
from jax.experimental import pallas as pl


def kernel(x, Wg, W1, b1, W2, b2):
    raise NotImplementedError("write your pallas kernel here")


# speedup vs baseline: 3.0805x; 3.0805x over previous
import jax
import jax.numpy as jnp
from jax.experimental import pallas as pl

_BT = 512


def _moe_body(x_ref, wg_ref, w1_ref, b1_ref, w2_ref, b2_ref, o_ref):
    E, FF, D = w2_ref.shape
    x = x_ref[...].astype(jnp.bfloat16)

    logits = jnp.dot(x, wg_ref[...], preferred_element_type=jnp.float32)
    logits = logits - jnp.max(logits, axis=-1, keepdims=True)
    eg = jnp.exp(logits)
    g = eg / jnp.sum(eg, axis=-1, keepdims=True)

    h = jnp.dot(x, w1_ref[...], preferred_element_type=jnp.float32)
    h = h + b1_ref[...]
    h = 0.5 * h * (1.0 + jax.lax.erf(h * 0.7071067811865476))

    acc = jnp.zeros_like(o_ref)
    for e in range(E):
        ge = g[:, e : e + 1]
        hs = (h[:, e * FF : (e + 1) * FF] * ge).astype(jnp.bfloat16)
        acc = acc + jnp.dot(hs, w2_ref[e], preferred_element_type=jnp.float32)
        acc = acc + ge * b2_ref[e][None, :]
    o_ref[...] = acc


def kernel(x, Wg, W1, b1, W2, b2):
    T, D = x.shape
    E, FF, _ = W1.shape
    wgp = Wg.T.astype(jnp.bfloat16)
    w1p = jnp.transpose(W1, (2, 0, 1)).reshape(D, E * FF).astype(jnp.bfloat16)
    b1p = b1.reshape(1, E * FF)
    w2p = jnp.transpose(W2, (0, 2, 1)).astype(jnp.bfloat16)

    return pl.pallas_call(
        _moe_body,
        grid=(T // _BT,),
        in_specs=[
            pl.BlockSpec((_BT, D), lambda i: (i, 0)),
            pl.BlockSpec((D, E), lambda i: (0, 0)),
            pl.BlockSpec((D, E * FF), lambda i: (0, 0)),
            pl.BlockSpec((1, E * FF), lambda i: (0, 0)),
            pl.BlockSpec((E, FF, D), lambda i: (0, 0, 0)),
            pl.BlockSpec((E, D), lambda i: (0, 0)),
        ],
        out_specs=pl.BlockSpec((_BT, D), lambda i: (i, 0)),
        out_shape=jax.ShapeDtypeStruct((T, D), jnp.float32),
    )(x, wgp, w1p, b1p, w2p, b2)


# speedup vs baseline: 3.0909x; 1.0034x over previous
import jax
import jax.numpy as jnp
from jax.experimental import pallas as pl

_BT = 512


def _moe_body(x_ref, wg_ref, w1_ref, b1_ref, w2_ref, b2_ref, o_ref):
    E, FF, D = w2_ref.shape
    x = x_ref[...].astype(jnp.bfloat16)

    logits = jnp.dot(x, wg_ref[...], preferred_element_type=jnp.float32)
    logits = logits - jnp.max(logits, axis=-1, keepdims=True)
    eg = jnp.exp(logits)
    g = eg / jnp.sum(eg, axis=-1, keepdims=True)

    h = jnp.dot(x, w1_ref[...], preferred_element_type=jnp.float32)
    h = (h + b1_ref[...]).astype(jnp.bfloat16)
    c0 = jnp.bfloat16(0.7978845608028654)
    c01 = jnp.bfloat16(0.7978845608028654 * 0.044715)
    t = jnp.tanh(h * (c0 + c01 * (h * h)))
    hg = h * (jnp.bfloat16(1.0) + t)

    acc = jnp.dot(g, b2_ref[...], preferred_element_type=jnp.float32)
    gh = (g * 0.5).astype(jnp.bfloat16)
    for e in range(E):
        hs = hg[:, e * FF : (e + 1) * FF] * gh[:, e : e + 1]
        acc = acc + jnp.dot(hs, w2_ref[e], preferred_element_type=jnp.float32)
    o_ref[...] = acc


def kernel(x, Wg, W1, b1, W2, b2):
    T, D = x.shape
    E, FF, _ = W1.shape
    wgp = Wg.T.astype(jnp.bfloat16)
    w1p = jnp.transpose(W1, (2, 0, 1)).reshape(D, E * FF).astype(jnp.bfloat16)
    b1p = b1.reshape(1, E * FF)
    w2p = jnp.transpose(W2, (0, 2, 1)).astype(jnp.bfloat16)

    return pl.pallas_call(
        _moe_body,
        grid=(T // _BT,),
        in_specs=[
            pl.BlockSpec((_BT, D), lambda i: (i, 0)),
            pl.BlockSpec((D, E), lambda i: (0, 0)),
            pl.BlockSpec((D, E * FF), lambda i: (0, 0)),
            pl.BlockSpec((1, E * FF), lambda i: (0, 0)),
            pl.BlockSpec((E, FF, D), lambda i: (0, 0, 0)),
            pl.BlockSpec((E, D), lambda i: (0, 0)),
        ],
        out_specs=pl.BlockSpec((_BT, D), lambda i: (i, 0)),
        out_shape=jax.ShapeDtypeStruct((T, D), jnp.float32),
    )(x, wgp, w1p, b1p, w2p, b2)
